# Initial kernel scaffold; baseline (speedup 1.0000x reference)
import jax
import jax.numpy as jnp
from jax.experimental import pallas as pl


def kernel(H, edge_index, gamma, beta, moving_mean, moving_var, W, b):
    raise NotImplementedError("write your pallas kernel here")


# speedup vs baseline: 6.4726x; 6.4726x over previous
import functools

import jax
import jax.numpy as jnp
from jax import lax
from jax.experimental import pallas as pl
from jax.experimental.pallas import tpu as pltpu
from jax.experimental.pallas import tpu_sc as plsc

N = 10000
E = 320000
D = 128
BN_EPS = 1e-3

NC = 2
NS = 16
NW = NC * NS
EPW = E // NW
EB = 128
NFULL = EPW // EB
REM = EPW - NFULL * EB
RPT = 624
RPT_LAST = N - RPT * (NS - 1)


def _agg_body(h_hbm, dst_hbm, src_hbm, out_hbm,
              idx_s, idx_d, rows, idx_sr, idx_dr, rows_r, acc, sem):
    c = lax.axis_index("c")
    s = lax.axis_index("s")
    wid = c * NS + s

    def zrow(i, _):
        for j in range(D // 16):
            rows[i, pl.ds(j * 16, 16)] = jnp.zeros((16,), jnp.float32)
        return 0
    lax.fori_loop(0, EB, zrow, 0)

    zbase = s * RPT
    for k in range((RPT + EB - 1) // EB):
        nrows = min(EB, RPT - k * EB)
        pltpu.sync_copy(rows.at[pl.ds(0, nrows)],
                        acc.at[pl.ds(zbase + k * EB, nrows)])

    extra = RPT_LAST - RPT

    @pl.when(s == NS - 1)
    def _zero_tail():
        pltpu.sync_copy(rows.at[pl.ds(0, extra)],
                        acc.at[pl.ds(RPT * NS, extra)])

    plsc.subcore_barrier()

    ebase = wid * EPW

    def body(i, _):
        off = ebase + i * EB
        pltpu.sync_copy(src_hbm.at[pl.ds(off, EB)], idx_s)
        pltpu.sync_copy(dst_hbm.at[pl.ds(off, EB)], idx_d)
        pltpu.async_copy(h_hbm.at[idx_s], rows, sem).wait()
        pltpu.sync_copy(rows, acc.at[idx_d], add=True)
        return 0
    lax.fori_loop(0, NFULL, body, 0)

    off = ebase + NFULL * EB
    pltpu.sync_copy(src_hbm.at[pl.ds(off, REM)], idx_sr)
    pltpu.sync_copy(dst_hbm.at[pl.ds(off, REM)], idx_dr)
    pltpu.async_copy(h_hbm.at[idx_sr], rows_r, sem).wait()
    pltpu.sync_copy(rows_r, acc.at[idx_dr], add=True)

    plsc.subcore_barrier()

    pltpu.sync_copy(acc.at[pl.ds(s * RPT, RPT)],
                    out_hbm.at[pl.ds(c * N + s * RPT, RPT)])

    @pl.when(s == NS - 1)
    def _drain_tail():
        pltpu.sync_copy(acc.at[pl.ds(RPT * NS, extra)],
                        out_hbm.at[pl.ds(c * N + RPT * NS, extra)])


def _make_agg():
    mesh = plsc.VectorSubcoreMesh(core_axis_name="c", subcore_axis_name="s")
    return pl.kernel(
        _agg_body,
        out_type=jax.ShapeDtypeStruct((NC * N, D), jnp.float32),
        mesh=mesh,
        scratch_types=[
            pltpu.VMEM((EB,), jnp.int32),
            pltpu.VMEM((EB,), jnp.int32),
            pltpu.VMEM((EB, D), jnp.float32),
            pltpu.VMEM((REM,), jnp.int32),
            pltpu.VMEM((REM,), jnp.int32),
            pltpu.VMEM((REM, D), jnp.float32),
            pltpu.VMEM_SHARED((N, D), jnp.float32),
            pltpu.SemaphoreType.DMA,
        ],
    )


ROWS_B = 400
GRID = N // ROWS_B


def _ffn_body(h_ref, p0_ref, p1_ref, scale_ref, shift_ref, w_ref, b_ref, o_ref):
    h = h_ref[...] + p0_ref[...] + p1_ref[...]
    x = h * scale_ref[...] + shift_ref[...]
    y = jnp.dot(x, w_ref[...], preferred_element_type=jnp.float32) + b_ref[...]
    z = 0.5 * y * (1.0 + lax.erf(y * (2.0 ** -0.5)))
    sq = jnp.sum(z * z, axis=1, keepdims=True)
    o_ref[...] = z * lax.rsqrt(jnp.maximum(sq, 1e-12))


def _ffn(H, p0, p1, scale, shift, W, b):
    row_spec = pl.BlockSpec((ROWS_B, D), lambda i: (i, 0))
    vec_spec = pl.BlockSpec((1, D), lambda i: (0, 0))
    return pl.pallas_call(
        _ffn_body,
        grid=(GRID,),
        in_specs=[row_spec, row_spec, row_spec, vec_spec, vec_spec,
                  pl.BlockSpec((D, D), lambda i: (0, 0)), vec_spec],
        out_specs=row_spec,
        out_shape=jax.ShapeDtypeStruct((N, D), jnp.float32),
    )(H, p0, p1, scale, shift, W, b)


@jax.jit
def kernel(H, edge_index, gamma, beta, moving_mean, moving_var, W, b):
    dst = edge_index[0]
    src = edge_index[1]
    partial = _make_agg()(H, dst, src)
    scale = gamma * lax.rsqrt(moving_var + BN_EPS)
    shift = beta - moving_mean * scale
    return _ffn(H, partial[:N], partial[N:],
                scale.reshape(1, D), shift.reshape(1, D), W, b.reshape(1, D))


# speedup vs baseline: 8.3975x; 1.2974x over previous
import functools

import jax
import jax.numpy as jnp
from jax import lax
from jax.experimental import pallas as pl
from jax.experimental.pallas import tpu as pltpu
from jax.experimental.pallas import tpu_sc as plsc

N = 10000
E = 320000
D = 128
BN_EPS = 1e-3

NC = 2
NS = 16
NW = NC * NS
EB = 128
NB = 78
EPW = NB * EB
NX = E - NW * EPW
K = 3
RPT = 624
RPT_LAST = N - RPT * (NS - 1)


def _agg_body(h_hbm, dst_hbm, src_hbm, out_hbm, *refs):
    rows = refs[0:K]
    idx_d = refs[K:2 * K]
    idx_s_big, idx_d_big, acc = refs[2 * K:2 * K + 3]
    gsem = refs[2 * K + 3:3 * K + 3]
    ssem = refs[3 * K + 3:4 * K + 3]

    c = lax.axis_index("c")
    s = lax.axis_index("s")
    wid = c * NS + s

    def zrow(i, _):
        for j in range(D // 16):
            rows[0][i, pl.ds(j * 16, 16)] = jnp.zeros((16,), jnp.float32)
        return 0
    lax.fori_loop(0, EB, zrow, 0)

    zbase = s * RPT
    for k in range((RPT + EB - 1) // EB):
        nrows = min(EB, RPT - k * EB)
        pltpu.sync_copy(rows[0].at[pl.ds(0, nrows)],
                        acc.at[pl.ds(zbase + k * EB, nrows)])

    extra = RPT_LAST - RPT

    @pl.when(s == NS - 1)
    def _zero_tail():
        pltpu.sync_copy(rows[0].at[pl.ds(0, extra)],
                        acc.at[pl.ds(RPT * NS, extra)])

    plsc.subcore_barrier()

    ebase = wid * EPW

    def body(g, _):
        base = ebase + g * (K * EB)
        pltpu.sync_copy(src_hbm.at[pl.ds(base, K * EB)], idx_s_big)
        pltpu.sync_copy(dst_hbm.at[pl.ds(base, K * EB)], idx_d_big)
        gd = []
        for u in range(K):
            for v in range(EB // 16):
                idx_d[u][pl.ds(v * 16, 16)] = (
                    idx_d_big[pl.ds(u * EB + v * 16, 16)])
            gd.append(pltpu.async_copy(
                h_hbm.at[idx_s_big.at[pl.ds(u * EB, EB)]],
                rows[u], gsem[u]))
        sd = []
        for u in range(K):
            gd[u].wait()
            sd.append(pltpu.async_copy(rows[u], acc.at[idx_d[u]],
                                       ssem[u], add=True))
        for u in range(K):
            sd[u].wait()
        return 0
    lax.fori_loop(0, NB // K, body, 0)

    @pl.when(wid < NX // EB)
    def _extra_batch():
        off = NW * EPW + wid * EB
        pltpu.sync_copy(src_hbm.at[pl.ds(off, EB)], idx_s_big.at[pl.ds(0, EB)])
        pltpu.sync_copy(dst_hbm.at[pl.ds(off, EB)], idx_d[0])
        pltpu.async_copy(h_hbm.at[idx_s_big.at[pl.ds(0, EB)]],
                         rows[0], gsem[0]).wait()
        pltpu.async_copy(rows[0], acc.at[idx_d[0]], ssem[0], add=True).wait()

    plsc.subcore_barrier()

    pltpu.sync_copy(acc.at[pl.ds(s * RPT, RPT)],
                    out_hbm.at[pl.ds(c * N + s * RPT, RPT)])

    @pl.when(s == NS - 1)
    def _drain_tail():
        pltpu.sync_copy(acc.at[pl.ds(RPT * NS, extra)],
                        out_hbm.at[pl.ds(c * N + RPT * NS, extra)])


def _make_agg():
    mesh = plsc.VectorSubcoreMesh(core_axis_name="c", subcore_axis_name="s")
    scratch = (
        [pltpu.VMEM((EB, D), jnp.float32)] * K +
        [pltpu.VMEM((EB,), jnp.int32)] * K +
        [pltpu.VMEM((K * EB,), jnp.int32),
         pltpu.VMEM((K * EB,), jnp.int32),
         pltpu.VMEM_SHARED((N, D), jnp.float32)] +
        [pltpu.SemaphoreType.DMA] * (2 * K)
    )
    return pl.kernel(
        _agg_body,
        out_type=jax.ShapeDtypeStruct((NC * N, D), jnp.float32),
        mesh=mesh,
        scratch_types=scratch,
    )


ROWS_B = 400
GRID = N // ROWS_B


def _ffn_body(h_ref, p0_ref, p1_ref, scale_ref, shift_ref, w_ref, b_ref, o_ref):
    h = h_ref[...] + p0_ref[...] + p1_ref[...]
    x = h * scale_ref[...] + shift_ref[...]
    y = jnp.dot(x, w_ref[...], preferred_element_type=jnp.float32) + b_ref[...]
    z = 0.5 * y * (1.0 + lax.erf(y * (2.0 ** -0.5)))
    sq = jnp.sum(z * z, axis=1, keepdims=True)
    o_ref[...] = z * lax.rsqrt(jnp.maximum(sq, 1e-12))


def _ffn(H, p0, p1, scale, shift, W, b):
    row_spec = pl.BlockSpec((ROWS_B, D), lambda i: (i, 0))
    vec_spec = pl.BlockSpec((1, D), lambda i: (0, 0))
    return pl.pallas_call(
        _ffn_body,
        grid=(GRID,),
        in_specs=[row_spec, row_spec, row_spec, vec_spec, vec_spec,
                  pl.BlockSpec((D, D), lambda i: (0, 0)), vec_spec],
        out_specs=row_spec,
        out_shape=jax.ShapeDtypeStruct((N, D), jnp.float32),
    )(H, p0, p1, scale, shift, W, b)


@jax.jit
def kernel(H, edge_index, gamma, beta, moving_mean, moving_var, W, b):
    dst = edge_index[0]
    src = edge_index[1]
    partial = _make_agg()(H, dst, src)
    scale = gamma * lax.rsqrt(moving_var + BN_EPS)
    shift = beta - moving_mean * scale
    return _ffn(H, partial[:N], partial[N:],
                scale.reshape(1, D), shift.reshape(1, D), W, b.reshape(1, D))


# speedup vs baseline: 9.8246x; 1.1699x over previous
import functools

import jax
import jax.numpy as jnp
from jax import lax
from jax.experimental import pallas as pl
from jax.experimental.pallas import tpu as pltpu
from jax.experimental.pallas import tpu_sc as plsc

N = 10000
E = 320000
D = 128
BN_EPS = 1e-3

NC = 2
NS = 16
NW = NC * NS
EB = 128
NB = 78
EPW = NB * EB
NX = E - NW * EPW
K = 3
RPT = 624
RPT_LAST = N - RPT * (NS - 1)


def _agg_body(h_hbm, dst_hbm, src_hbm, out_hbm, *refs):
    rows = refs[0:K]
    idx_d = refs[K:2 * K]
    idx_s_big = refs[2 * K:2 * K + 2]
    idx_d_big = refs[2 * K + 2:2 * K + 4]
    acc = refs[2 * K + 4]
    gsem = refs[2 * K + 5:3 * K + 5]
    ssem = refs[3 * K + 5:4 * K + 5]

    c = lax.axis_index("c")
    s = lax.axis_index("s")
    wid = c * NS + s

    def zrow(i, _):
        for j in range(D // 16):
            rows[0][i, pl.ds(j * 16, 16)] = jnp.zeros((16,), jnp.float32)
        return 0
    lax.fori_loop(0, EB, zrow, 0)

    zbase = s * RPT
    for k in range((RPT + EB - 1) // EB):
        nrows = min(EB, RPT - k * EB)
        pltpu.sync_copy(rows[0].at[pl.ds(0, nrows)],
                        acc.at[pl.ds(zbase + k * EB, nrows)])

    extra = RPT_LAST - RPT

    @pl.when(s == NS - 1)
    def _zero_tail():
        pltpu.sync_copy(rows[0].at[pl.ds(0, extra)],
                        acc.at[pl.ds(RPT * NS, extra)])

    plsc.subcore_barrier()

    ebase = wid * EPW

    def one_body(g, p, first):
        base = ebase + g * (K * EB)
        pltpu.sync_copy(src_hbm.at[pl.ds(base, K * EB)], idx_s_big[p])
        pltpu.sync_copy(dst_hbm.at[pl.ds(base, K * EB)], idx_d_big[p])
        gd = []
        for u in range(K):
            if not first:
                pltpu.make_async_copy(rows[u], acc.at[idx_d[u]],
                                      ssem[u]).wait()
            for v in range(EB // 16):
                idx_d[u][pl.ds(v * 16, 16)] = (
                    idx_d_big[p][pl.ds(u * EB + v * 16, 16)])
            gd.append(pltpu.async_copy(
                h_hbm.at[idx_s_big[p].at[pl.ds(u * EB, EB)]],
                rows[u], gsem[u]))
        for u in range(K):
            gd[u].wait()
            pltpu.async_copy(rows[u], acc.at[idx_d[u]], ssem[u], add=True)

    one_body(0, 0, True)

    def body(t, _):
        one_body(2 * t + 1, 1, False)
        one_body(2 * t + 2, 0, False)
        return 0
    lax.fori_loop(0, (NB // K - 1) // 2, body, 0)

    one_body(NB // K - 1, 1, False)

    for u in range(K):
        pltpu.make_async_copy(rows[u], acc.at[idx_d[u]], ssem[u]).wait()

    @pl.when(wid < NX // EB)
    def _extra_batch():
        off = NW * EPW + wid * EB
        pltpu.sync_copy(src_hbm.at[pl.ds(off, EB)],
                        idx_s_big[0].at[pl.ds(0, EB)])
        pltpu.sync_copy(dst_hbm.at[pl.ds(off, EB)], idx_d[0])
        pltpu.async_copy(h_hbm.at[idx_s_big[0].at[pl.ds(0, EB)]],
                         rows[0], gsem[0]).wait()
        pltpu.async_copy(rows[0], acc.at[idx_d[0]], ssem[0], add=True).wait()

    plsc.subcore_barrier()

    pltpu.sync_copy(acc.at[pl.ds(s * RPT, RPT)],
                    out_hbm.at[pl.ds(c * N + s * RPT, RPT)])

    @pl.when(s == NS - 1)
    def _drain_tail():
        pltpu.sync_copy(acc.at[pl.ds(RPT * NS, extra)],
                        out_hbm.at[pl.ds(c * N + RPT * NS, extra)])


def _make_agg():
    mesh = plsc.VectorSubcoreMesh(core_axis_name="c", subcore_axis_name="s")
    scratch = (
        [pltpu.VMEM((EB, D), jnp.float32)] * K +
        [pltpu.VMEM((EB,), jnp.int32)] * K +
        [pltpu.VMEM((K * EB,), jnp.int32)] * 2 +
        [pltpu.VMEM((K * EB,), jnp.int32)] * 2 +
        [pltpu.VMEM_SHARED((N, D), jnp.float32)] +
        [pltpu.SemaphoreType.DMA] * (2 * K)
    )
    return pl.kernel(
        _agg_body,
        out_type=jax.ShapeDtypeStruct((NC * N, D), jnp.float32),
        mesh=mesh,
        scratch_types=scratch,
    )


ROWS_B = 400
GRID = N // ROWS_B


def _ffn_body(h_ref, p0_ref, p1_ref, scale_ref, shift_ref, w_ref, b_ref, o_ref):
    h = h_ref[...] + p0_ref[...] + p1_ref[...]
    x = h * scale_ref[...] + shift_ref[...]
    y = jnp.dot(x, w_ref[...], preferred_element_type=jnp.float32) + b_ref[...]
    z = 0.5 * y * (1.0 + lax.erf(y * (2.0 ** -0.5)))
    sq = jnp.sum(z * z, axis=1, keepdims=True)
    o_ref[...] = z * lax.rsqrt(jnp.maximum(sq, 1e-12))


def _ffn(H, p0, p1, scale, shift, W, b):
    row_spec = pl.BlockSpec((ROWS_B, D), lambda i: (i, 0))
    vec_spec = pl.BlockSpec((1, D), lambda i: (0, 0))
    return pl.pallas_call(
        _ffn_body,
        grid=(GRID,),
        in_specs=[row_spec, row_spec, row_spec, vec_spec, vec_spec,
                  pl.BlockSpec((D, D), lambda i: (0, 0)), vec_spec],
        out_specs=row_spec,
        out_shape=jax.ShapeDtypeStruct((N, D), jnp.float32),
    )(H, p0, p1, scale, shift, W, b)


@jax.jit
def kernel(H, edge_index, gamma, beta, moving_mean, moving_var, W, b):
    dst = edge_index[0]
    src = edge_index[1]
    partial = _make_agg()(H, dst, src)
    scale = gamma * lax.rsqrt(moving_var + BN_EPS)
    shift = beta - moving_mean * scale
    return _ffn(H, partial[:N], partial[N:],
                scale.reshape(1, D), shift.reshape(1, D), W, b.reshape(1, D))


# speedup vs baseline: 9.9421x; 1.0120x over previous
import functools

import jax
import jax.numpy as jnp
from jax import lax
from jax.experimental import pallas as pl
from jax.experimental.pallas import tpu as pltpu
from jax.experimental.pallas import tpu_sc as plsc

N = 10000
E = 320000
D = 128
BN_EPS = 1e-3

NC = 2
NS = 16
NW = NC * NS
EB = 128
NB = 78
EPW = NB * EB
NX = E - NW * EPW
K = 3
RPT = 624
RPT_LAST = N - RPT * (NS - 1)


def _agg_body(h_hbm, dst_hbm, src_hbm, out_hbm, *refs):
    rows = refs[0:K]
    idx_d = refs[K:2 * K]
    idx_s_big = refs[2 * K:2 * K + 2]
    idx_d_big = refs[2 * K + 2:2 * K + 4]
    acc = refs[2 * K + 4]
    gsem = refs[2 * K + 5:3 * K + 5]
    ssem = refs[3 * K + 5:4 * K + 5]

    c = lax.axis_index("c")
    s = lax.axis_index("s")
    wid = c * NS + s

    def zrow(i, _):
        for j in range(D // 16):
            rows[0][i, pl.ds(j * 16, 16)] = jnp.zeros((16,), jnp.float32)
        return 0
    lax.fori_loop(0, EB, zrow, 0)

    zbase = s * RPT
    for k in range((RPT + EB - 1) // EB):
        nrows = min(EB, RPT - k * EB)
        pltpu.sync_copy(rows[0].at[pl.ds(0, nrows)],
                        acc.at[pl.ds(zbase + k * EB, nrows)])

    extra = RPT_LAST - RPT

    @pl.when(s == NS - 1)
    def _zero_tail():
        pltpu.sync_copy(rows[0].at[pl.ds(0, extra)],
                        acc.at[pl.ds(RPT * NS, extra)])

    plsc.subcore_barrier()

    ebase = wid * EPW

    def one_body(g, p, first):
        base = ebase + g * (K * EB)
        pltpu.sync_copy(src_hbm.at[pl.ds(base, K * EB)], idx_s_big[p])
        pltpu.sync_copy(dst_hbm.at[pl.ds(base, K * EB)], idx_d_big[p])
        gd = []
        for u in range(K):
            if not first:
                pltpu.make_async_copy(rows[u], acc.at[idx_d[u]],
                                      ssem[u]).wait()
            for v in range(EB // 16):
                idx_d[u][pl.ds(v * 16, 16)] = (
                    idx_d_big[p][pl.ds(u * EB + v * 16, 16)])
            gd.append(pltpu.async_copy(
                h_hbm.at[idx_s_big[p].at[pl.ds(u * EB, EB)]],
                rows[u], gsem[u]))
        for u in range(K):
            gd[u].wait()
            pltpu.async_copy(rows[u], acc.at[idx_d[u]], ssem[u], add=True)

    one_body(0, 0, True)

    def body(t, _):
        one_body(2 * t + 1, 1, False)
        one_body(2 * t + 2, 0, False)
        return 0
    lax.fori_loop(0, (NB // K - 1) // 2, body, 0)

    one_body(NB // K - 1, 1, False)

    for u in range(K):
        pltpu.make_async_copy(rows[u], acc.at[idx_d[u]], ssem[u]).wait()

    @pl.when(wid < NX // EB)
    def _extra_batch():
        off = NW * EPW + wid * EB
        pltpu.sync_copy(src_hbm.at[pl.ds(off, EB)],
                        idx_s_big[0].at[pl.ds(0, EB)])
        pltpu.sync_copy(dst_hbm.at[pl.ds(off, EB)], idx_d[0])
        pltpu.async_copy(h_hbm.at[idx_s_big[0].at[pl.ds(0, EB)]],
                         rows[0], gsem[0]).wait()
        pltpu.async_copy(rows[0], acc.at[idx_d[0]], ssem[0], add=True).wait()

    plsc.subcore_barrier()

    pltpu.sync_copy(acc.at[pl.ds(s * RPT, RPT)],
                    out_hbm.at[pl.ds(c * N + s * RPT, RPT)])

    @pl.when(s == NS - 1)
    def _drain_tail():
        pltpu.sync_copy(acc.at[pl.ds(RPT * NS, extra)],
                        out_hbm.at[pl.ds(c * N + RPT * NS, extra)])


def _make_agg():
    mesh = plsc.VectorSubcoreMesh(core_axis_name="c", subcore_axis_name="s")
    scratch = (
        [pltpu.VMEM((EB, D), jnp.float32)] * K +
        [pltpu.VMEM((EB,), jnp.int32)] * K +
        [pltpu.VMEM((K * EB,), jnp.int32)] * 2 +
        [pltpu.VMEM((K * EB,), jnp.int32)] * 2 +
        [pltpu.VMEM_SHARED((N, D), jnp.float32)] +
        [pltpu.SemaphoreType.DMA] * (2 * K)
    )
    return pl.kernel(
        _agg_body,
        out_type=jax.ShapeDtypeStruct((NC * N, D), jnp.float32),
        mesh=mesh,
        scratch_types=scratch,
    )


ROWS_B = 400
GRID = N // ROWS_B


def _ffn_body(h_ref, p0_ref, p1_ref, scale_ref, shift_ref, w_ref, b_ref, o_ref):
    h = h_ref[...] + p0_ref[...] + p1_ref[...]
    x = h * scale_ref[...] + shift_ref[...]
    y = jnp.dot(x, w_ref[...], preferred_element_type=jnp.float32) + b_ref[...]
    z = 0.5 * y * (1.0 + lax.erf(y * (2.0 ** -0.5)))
    sq = jnp.sum(z * z, axis=1, keepdims=True)
    o_ref[...] = z * lax.rsqrt(jnp.maximum(sq, 1e-12))


def _ffn(H, partial, scale, shift, W, b):
    row_spec = pl.BlockSpec((ROWS_B, D), lambda i: (i, 0))
    p0_spec = pl.BlockSpec((ROWS_B, D), lambda i: (i, 0))
    p1_spec = pl.BlockSpec((ROWS_B, D), lambda i: (i + GRID, 0))
    vec_spec = pl.BlockSpec((1, D), lambda i: (0, 0))
    return pl.pallas_call(
        _ffn_body,
        grid=(GRID,),
        in_specs=[row_spec, p0_spec, p1_spec, vec_spec, vec_spec,
                  pl.BlockSpec((D, D), lambda i: (0, 0)), vec_spec],
        out_specs=row_spec,
        out_shape=jax.ShapeDtypeStruct((N, D), jnp.float32),
    )(H, partial, partial, scale, shift, W, b)


@jax.jit
def kernel(H, edge_index, gamma, beta, moving_mean, moving_var, W, b):
    dst = edge_index[0]
    src = edge_index[1]
    partial = _make_agg()(H, dst, src)
    scale = gamma * lax.rsqrt(moving_var + BN_EPS)
    shift = beta - moving_mean * scale
    return _ffn(H, partial,
                scale.reshape(1, D), shift.reshape(1, D), W, b.reshape(1, D))
